# Initial kernel scaffold; baseline (speedup 1.0000x reference)
import jax
import jax.numpy as jnp
from jax.experimental import pallas as pl


def kernel(X, y_values, x_min, x_max):
    raise NotImplementedError("write your pallas kernel here")


# speedup vs baseline: 361.5023x; 361.5023x over previous
import functools

import jax
import jax.numpy as jnp
from jax import lax
from jax.experimental import pallas as pl
from jax.experimental.pallas import tpu as pltpu
from jax.experimental.pallas import tpu_sc as plsc

_NC = 2
_NS = 16
_L = 16
_NW = _NC * _NS


def _make_fq(E, C, S, CH, NCH):
    EW = E // _NW
    TAB = C * S

    mesh = plsc.VectorSubcoreMesh(core_axis_name="c", subcore_axis_name="s")

    @functools.partial(
        pl.kernel,
        mesh=mesh,
        out_type=jax.ShapeDtypeStruct((E,), jnp.float32),
        compiler_params=pltpu.CompilerParams(needs_layout_passes=False),
        scratch_types=[
            pltpu.VMEM((TAB,), jnp.float32),
            pltpu.VMEM((_L,), jnp.float32),
            pltpu.VMEM((_L,), jnp.float32),
            pltpu.VMEM((CH,), jnp.float32),
            pltpu.VMEM((CH,), jnp.float32),
            pltpu.VMEM((CH,), jnp.float32),
            pltpu.VMEM((CH,), jnp.float32),
            pltpu.SemaphoreType.DMA,
            pltpu.SemaphoreType.DMA,
            pltpu.SemaphoreType.DMA,
            pltpu.SemaphoreType.DMA,
        ],
    )
    def fq(x_hbm, tab_hbm, xmn_hbm, xmx_hbm, o_hbm,
           tab_v, xmn_v, xmx_v, a0, a1, b0, b1, si0, si1, so0, so1):
        wid = lax.axis_index("s") * _NC + lax.axis_index("c")
        base = wid * EW

        pltpu.sync_copy(tab_hbm, tab_v)
        pltpu.sync_copy(xmn_hbm, xmn_v)
        pltpu.sync_copy(xmx_hbm, xmx_v)

        ins = (a0, a1)
        outs = (b0, b1)
        sin = (si0, si1)
        sout = (so0, so1)

        def in_copy(k):
            return pltpu.make_async_copy(
                x_hbm.at[pl.ds(base + k * CH, CH)], ins[k % 2], sin[k % 2])

        def out_copy(k):
            return pltpu.make_async_copy(
                outs[k % 2], o_hbm.at[pl.ds(base + k * CH, CH)], sout[k % 2])

        xmn = xmn_v[...]
        scl = float(S - 1) / (xmx_v[...] - xmn)
        lane_off = lax.iota(jnp.int32, _L) * S
        tmax = float(S - 1)
        imax = S - 2

        def chunk(src, dst):
            def body(i, c):
                x = src[pl.ds(i * _L, _L)]
                t = (x - xmn) * scl
                t = jnp.minimum(jnp.maximum(t, 0.0), tmax)
                i0 = t.astype(jnp.int32)
                i0 = jnp.where(i0.astype(jnp.float32) > t, i0 - 1, i0)
                i0 = jnp.minimum(jnp.maximum(i0, 0), imax)
                fr = t - i0.astype(jnp.float32)
                idx = lane_off + i0
                y0 = plsc.load_gather(tab_v, [idx])
                y1 = plsc.load_gather(tab_v, [idx + 1])
                dst[pl.ds(i * _L, _L)] = y0 * (1.0 - fr) + y1 * fr
                return c
            lax.fori_loop(0, CH // _L, body, 0)

        in_copy(0).start()
        for k in range(NCH):
            if k + 1 < NCH:
                in_copy(k + 1).start()
            in_copy(k).wait()
            if k >= 2:
                out_copy(k - 2).wait()
            chunk(ins[k % 2], outs[k % 2])
            out_copy(k).start()
        out_copy(NCH - 2).wait()
        out_copy(NCH - 1).wait()

    return fq


def kernel(X, y_values, x_min, x_max):
    N, C = X.shape
    S = y_values.shape[1]
    E = N * C
    EW = E // _NW
    CH = 16384
    while EW % CH:
        CH //= 2
    NCH = EW // CH
    fq = _make_fq(E, C, S, CH, NCH)
    out = fq(X.reshape(E), y_values.reshape(C * S), x_min, x_max)
    return out.reshape(N, C)


# speedup vs baseline: 394.2045x; 1.0905x over previous
import functools

import jax
import jax.numpy as jnp
from jax import lax
from jax.experimental import pallas as pl
from jax.experimental.pallas import tpu as pltpu
from jax.experimental.pallas import tpu_sc as plsc

_NC = 2
_NS = 16
_L = 16
_NW = _NC * _NS


def _make_fq(E, C, S, CH, NCH):
    EW = E // _NW
    TAB = C * S

    mesh = plsc.VectorSubcoreMesh(core_axis_name="c", subcore_axis_name="s")

    @functools.partial(
        pl.kernel,
        mesh=mesh,
        out_type=jax.ShapeDtypeStruct((E,), jnp.float32),
        compiler_params=pltpu.CompilerParams(needs_layout_passes=False),
        scratch_types=[
            pltpu.VMEM((TAB,), jnp.float32),
            pltpu.VMEM((_L,), jnp.float32),
            pltpu.VMEM((_L,), jnp.float32),
            pltpu.VMEM((CH,), jnp.float32),
            pltpu.VMEM((CH,), jnp.float32),
            pltpu.VMEM((CH,), jnp.float32),
            pltpu.VMEM((CH,), jnp.float32),
            pltpu.SemaphoreType.DMA,
            pltpu.SemaphoreType.DMA,
            pltpu.SemaphoreType.DMA,
            pltpu.SemaphoreType.DMA,
        ],
    )
    def fq(x_hbm, tab_hbm, xmn_hbm, xmx_hbm, o_hbm,
           tab_v, xmn_v, xmx_v, a0, a1, b0, b1, si0, si1, so0, so1):
        wid = lax.axis_index("s") * _NC + lax.axis_index("c")
        base = wid * EW

        pltpu.sync_copy(tab_hbm, tab_v)
        pltpu.sync_copy(xmn_hbm, xmn_v)
        pltpu.sync_copy(xmx_hbm, xmx_v)

        ins = (a0, a1)
        outs = (b0, b1)
        sin = (si0, si1)
        sout = (so0, so1)

        def in_copy(k):
            return pltpu.make_async_copy(
                x_hbm.at[pl.ds(base + k * CH, CH)], ins[k % 2], sin[k % 2])

        def out_copy(k):
            return pltpu.make_async_copy(
                outs[k % 2], o_hbm.at[pl.ds(base + k * CH, CH)], sout[k % 2])

        xmn = xmn_v[...]
        scl = float(S - 1) / (xmx_v[...] - xmn)
        lane_off = lax.iota(jnp.int32, _L) * S
        tmax = float(S - 1)
        imax = S - 2

        def chunk(src, dst):
            @plsc.parallel_loop(0, CH // _L, unroll=8)
            def body(i):
                x = src[pl.ds(i * _L, _L)]
                t = (x - xmn) * scl
                t = jnp.minimum(jnp.maximum(t, 0.0), tmax)
                i0 = t.astype(jnp.int32)
                i0f = i0.astype(jnp.float32)
                up = i0f > t
                i0 = jnp.where(up, i0 - 1, i0)
                i0f = jnp.where(up, i0f - 1.0, i0f)
                hi = i0 > imax
                i0 = jnp.where(hi, imax, i0)
                i0f = jnp.where(hi, float(imax), i0f)
                fr = t - i0f
                idx = lane_off + i0
                y0 = plsc.load_gather(tab_v, [idx])
                y1 = plsc.load_gather(tab_v, [idx + 1])
                dst[pl.ds(i * _L, _L)] = y0 * (1.0 - fr) + y1 * fr

        in_copy(0).start()
        for k in range(NCH):
            if k + 1 < NCH:
                in_copy(k + 1).start()
            in_copy(k).wait()
            if k >= 2:
                out_copy(k - 2).wait()
            chunk(ins[k % 2], outs[k % 2])
            out_copy(k).start()
        out_copy(NCH - 2).wait()
        out_copy(NCH - 1).wait()

    return fq


def kernel(X, y_values, x_min, x_max):
    N, C = X.shape
    S = y_values.shape[1]
    E = N * C
    EW = E // _NW
    CH = 16384
    while EW % CH:
        CH //= 2
    NCH = EW // CH
    fq = _make_fq(E, C, S, CH, NCH)
    out = fq(X.reshape(E), y_values.reshape(C * S), x_min, x_max)
    return out.reshape(N, C)


# speedup vs baseline: 2836.4422x; 7.1954x over previous
import functools

import jax
import jax.numpy as jnp
from jax import lax
from jax.experimental import pallas as pl
from jax.experimental.pallas import tpu as pltpu
from jax.experimental.pallas import tpu_sc as plsc

_NC = 2
_NS = 16
_L = 16
_NW = _NC * _NS


def _make_fq(N, C, S, KB, NCH):
    TAB = C * S
    NB = N // 128
    HB = NB // 2
    CHI = C // 8

    mesh = plsc.VectorSubcoreMesh(core_axis_name="c", subcore_axis_name="s")

    @functools.partial(
        pl.kernel,
        mesh=mesh,
        out_type=jax.ShapeDtypeStruct((CHI, NB, 8, 128), jnp.float32),
        compiler_params=pltpu.CompilerParams(needs_layout_passes=False),
        scratch_types=[
            pltpu.VMEM((TAB,), jnp.float32),
            pltpu.VMEM((_L,), jnp.float32),
            pltpu.VMEM((_L,), jnp.float32),
            pltpu.VMEM((KB, 128), jnp.float32),
            pltpu.VMEM((KB, 128), jnp.float32),
            pltpu.VMEM((KB, 128), jnp.float32),
            pltpu.VMEM((KB, 128), jnp.float32),
            pltpu.SemaphoreType.DMA,
            pltpu.SemaphoreType.DMA,
            pltpu.SemaphoreType.DMA,
            pltpu.SemaphoreType.DMA,
        ],
    )
    def fq(x_hbm, tab_hbm, xmn_hbm, xmx_hbm, o_hbm,
           tab_v, xmn_v, xmx_v, a0, a1, b0, b1, si0, si1, so0, so1):
        wid = lax.axis_index("s") * _NC + lax.axis_index("c")
        col = wid // 2
        half = wid % 2
        chi = col // 8
        clo = col % 8
        blk0 = half * HB

        pltpu.sync_copy(tab_hbm, tab_v)
        pltpu.sync_copy(xmn_hbm, xmn_v)
        pltpu.sync_copy(xmx_hbm, xmx_v)

        ins = (a0, a1)
        outs = (b0, b1)
        sin = (si0, si1)
        sout = (so0, so1)

        def in_copy(k, p):
            return pltpu.make_async_copy(
                x_hbm.at[chi, pl.ds(blk0 + k * KB, KB), clo, :],
                ins[p], sin[p])

        def out_copy(k, p):
            return pltpu.make_async_copy(
                outs[p], o_hbm.at[chi, pl.ds(blk0 + k * KB, KB), clo, :],
                sout[p])

        cvec = jnp.broadcast_to(col, (_L,)).astype(jnp.int32)
        xmn = plsc.load_gather(xmn_v, [cvec])
        xmx = plsc.load_gather(xmx_v, [cvec])
        scl = float(S - 1) / (xmx - xmn)
        tab_base = cvec * S
        tmax = float(S - 1)
        bmax = float(S - 2)

        def chunk(src, dst):
            @plsc.parallel_loop(0, KB, unroll=2)
            def body(blk):
                for j in range(8):
                    x = src[blk, pl.ds(j * _L, _L)]
                    t = (x - xmn) * scl
                    t = jnp.minimum(jnp.maximum(t, 0.0), tmax)
                    i0 = jnp.minimum(t, bmax).astype(jnp.int32)
                    fr = t - i0.astype(jnp.float32)
                    idx = tab_base + i0
                    y0 = plsc.load_gather(tab_v, [idx])
                    y1 = plsc.load_gather(tab_v, [idx + 1])
                    dst[blk, pl.ds(j * _L, _L)] = y0 * (1.0 - fr) + y1 * fr

        in_copy(0, 0).start()
        in_copy(1, 1).start()
        def outer(kk, carry):
            for p in (0, 1):
                k = kk * 2 + p
                in_copy(k, p).wait()

                @pl.when(kk >= 1)
                def _():
                    out_copy(k - 2, p).wait()

                chunk(ins[p], outs[p])
                out_copy(k, p).start()

                @pl.when(kk < NCH // 2 - 1)
                def _():
                    in_copy(k + 2, p).start()
            return carry
        lax.fori_loop(0, NCH // 2, outer, 0)
        out_copy(NCH - 2, 0).wait()
        out_copy(NCH - 1, 1).wait()

    return fq


def kernel(X, y_values, x_min, x_max):
    N, C = X.shape
    S = y_values.shape[1]
    KB = 128
    HB = N // 128 // 2
    while HB % KB:
        KB //= 2
    NCH = HB // KB
    fq = _make_fq(N, C, S, KB, NCH)
    x4 = X.reshape(N // 128, 128, C // 8, 8).transpose(2, 0, 3, 1)
    out4 = fq(x4, y_values.reshape(C * S), x_min, x_max)
    return out4.transpose(1, 3, 0, 2).reshape(N, C)


# speedup vs baseline: 2934.5264x; 1.0346x over previous
import functools

import jax
import jax.numpy as jnp
from jax import lax
from jax.experimental import pallas as pl
from jax.experimental.pallas import tpu as pltpu
from jax.experimental.pallas import tpu_sc as plsc

_NC = 2
_NS = 16
_L = 16
_NW = _NC * _NS


def _make_fq(N, C, S, KB, NCH):
    TAB = C * S
    NB = N // 128
    HB = NB // 2
    CHI = C // 8

    mesh = plsc.VectorSubcoreMesh(core_axis_name="c", subcore_axis_name="s")

    @functools.partial(
        pl.kernel,
        mesh=mesh,
        out_type=jax.ShapeDtypeStruct((CHI, NB, 8, 128), jnp.float32),
        compiler_params=pltpu.CompilerParams(needs_layout_passes=False),
        scratch_types=[
            pltpu.VMEM((TAB,), jnp.float32),
            pltpu.VMEM((TAB,), jnp.float32),
            pltpu.VMEM((_L,), jnp.float32),
            pltpu.VMEM((_L,), jnp.float32),
            pltpu.VMEM((KB, 128), jnp.float32),
            pltpu.VMEM((KB, 128), jnp.float32),
            pltpu.VMEM((KB, 128), jnp.float32),
            pltpu.VMEM((KB, 128), jnp.float32),
            pltpu.SemaphoreType.DMA,
            pltpu.SemaphoreType.DMA,
            pltpu.SemaphoreType.DMA,
            pltpu.SemaphoreType.DMA,
        ],
    )
    def fq(x_hbm, tab_hbm, tab1_hbm, xmn_hbm, xmx_hbm, o_hbm,
           tab_v, tab1_v, xmn_v, xmx_v, a0, a1, b0, b1, si0, si1, so0, so1):
        wid = lax.axis_index("s") * _NC + lax.axis_index("c")
        col = wid // 2
        half = wid % 2
        chi = col // 8
        clo = col % 8
        blk0 = half * HB

        pltpu.sync_copy(tab_hbm, tab_v)
        pltpu.sync_copy(tab1_hbm, tab1_v)
        pltpu.sync_copy(xmn_hbm, xmn_v)
        pltpu.sync_copy(xmx_hbm, xmx_v)

        ins = (a0, a1)
        outs = (b0, b1)
        sin = (si0, si1)
        sout = (so0, so1)

        def in_copy(k, p):
            return pltpu.make_async_copy(
                x_hbm.at[chi, pl.ds(blk0 + k * KB, KB), clo, :],
                ins[p], sin[p])

        def out_copy(k, p):
            return pltpu.make_async_copy(
                outs[p], o_hbm.at[chi, pl.ds(blk0 + k * KB, KB), clo, :],
                sout[p])

        cvec = jnp.broadcast_to(col, (_L,)).astype(jnp.int32)
        xmn = plsc.load_gather(xmn_v, [cvec])
        xmx = plsc.load_gather(xmx_v, [cvec])
        scl = float(S - 1) / (xmx - xmn)
        tab_base = cvec * S
        tmax = float(S - 1)
        bmax = float(S - 2)

        def chunk(src, dst):
            @plsc.parallel_loop(0, KB, unroll=4)
            def body(blk):
                for j in range(8):
                    x = src[blk, pl.ds(j * _L, _L)]
                    t = (x - xmn) * scl
                    t = jnp.minimum(jnp.maximum(t, 0.0), tmax)
                    i0 = jnp.minimum(t, bmax).astype(jnp.int32)
                    fr = t - i0.astype(jnp.float32)
                    idx = tab_base + i0
                    y0 = plsc.load_gather(tab_v, [idx])
                    y1 = plsc.load_gather(tab1_v, [idx])
                    dst[blk, pl.ds(j * _L, _L)] = y0 + fr * (y1 - y0)

        in_copy(0, 0).start()
        in_copy(1, 1).start()
        def outer(kk, carry):
            for p in (0, 1):
                k = kk * 2 + p
                in_copy(k, p).wait()

                @pl.when(kk >= 1)
                def _():
                    out_copy(k - 2, p).wait()

                chunk(ins[p], outs[p])
                out_copy(k, p).start()

                @pl.when(kk < NCH // 2 - 1)
                def _():
                    in_copy(k + 2, p).start()
            return carry
        lax.fori_loop(0, NCH // 2, outer, 0)
        out_copy(NCH - 2, 0).wait()
        out_copy(NCH - 1, 1).wait()

    return fq


def kernel(X, y_values, x_min, x_max):
    N, C = X.shape
    S = y_values.shape[1]
    KB = 128
    HB = N // 128 // 2
    while HB % KB:
        KB //= 2
    NCH = HB // KB
    fq = _make_fq(N, C, S, KB, NCH)
    x4 = X.reshape(N // 128, 128, C // 8, 8).transpose(2, 0, 3, 1)
    yv1 = jnp.concatenate([y_values[:, 1:], y_values[:, -1:]], axis=1)
    out4 = fq(x4, y_values.reshape(C * S), yv1.reshape(C * S), x_min, x_max)
    return out4.transpose(1, 3, 0, 2).reshape(N, C)
